# baseline (speedup 1.0000x reference)
import functools

import jax
import jax.numpy as jnp
from jax import lax
from jax.experimental import pallas as pl
from jax.experimental.pallas import tpu as pltpu
from jax.experimental.pallas import tpu_sc as plsc

B, R, NUM_NUMERICAL = 4096, 20, 16
N = B * R
NC, NS = 2, 16
NW = NC * NS
ROWS_PER_W = N // NW
CHUNK = 128
NCH = ROWS_PER_W // CHUNK
NBUF = 2

NTAB = 6
DIMS_LIST = (16, 32, 32, 16, 16, 16)
COL0 = (0, 16, 48, 80, 96, 112)

TB = 128
TN = TB * R
GRID = B // TB


def _sc_body(idx_hbm, tab0, tab1, tab2, tab3, tab4, tab5, out,
             idx_v, b0, b1,
             sg0, sg1, sa0, sa1, so0, so1):
    wid = lax.axis_index("s") * NC + lax.axis_index("c")
    base = wid * ROWS_PER_W
    tabs = (tab0, tab1, tab2, tab3, tab4, tab5)
    bufs = (b0, b1)
    sgs = (sg0, sg1)
    sas = (sa0, sa1)
    sos = (so0, so1)
    pltpu.sync_copy(idx_hbm.at[wid], idx_v)

    def iter_body(i, _):
        for k in range(NBUF):
            j = i * NBUF + k

            @pl.when(j >= NBUF)
            def _(k=k):
                pltpu.make_async_copy(bufs[k],
                                      out.at[pl.ds(base, CHUNK)],
                                      sos[k]).wait()

            pltpu.async_copy(tabs[0].at[idx_v.at[0, j]], bufs[k], sgs[k])
        for k in range(NBUF):
            j = i * NBUF + k
            pltpu.make_async_copy(tabs[0].at[idx_v.at[0, 0]], bufs[k],
                                  sgs[k]).wait()
            for t in range(1, NTAB):
                pltpu.async_copy(tabs[t].at[idx_v.at[t, j]], bufs[k],
                                 sas[k], add=True)
        for k in range(NBUF):
            j = i * NBUF + k
            for t in range(1, NTAB):
                pltpu.make_async_copy(tabs[t].at[idx_v.at[t, 0]], bufs[k],
                                      sas[k]).wait()
            pltpu.async_copy(bufs[k],
                             out.at[pl.ds(base + j * CHUNK, CHUNK)],
                             sos[k])
        return 0

    lax.fori_loop(0, NCH // NBUF, iter_body, 0)
    for k in range(NBUF):
        pltpu.make_async_copy(bufs[k], out.at[pl.ds(base, CHUNK)],
                              sos[k]).wait()


@functools.partial(
    pl.kernel,
    out_type=jax.ShapeDtypeStruct((N, 128), jnp.float32),
    mesh=plsc.VectorSubcoreMesh(core_axis_name="c", subcore_axis_name="s",
                                num_cores=NC, num_subcores=NS),
    compiler_params=pltpu.CompilerParams(use_tc_tiling_on_sc=True),
    scratch_types=[
        pltpu.VMEM((NTAB, NCH, CHUNK), jnp.int32),
        pltpu.VMEM((CHUNK, 128), jnp.float32),
        pltpu.VMEM((CHUNK, 128), jnp.float32),
        pltpu.SemaphoreType.DMA,
        pltpu.SemaphoreType.DMA,
        pltpu.SemaphoreType.DMA,
        pltpu.SemaphoreType.DMA,
        pltpu.SemaphoreType.DMA,
        pltpu.SemaphoreType.DMA,
    ],
)
def _sc_gather(*args):
    _sc_body(*args)


def _mlp_body(emb, xn, w1e, w1n, b1r, w2, b2r, out):
    h = jnp.maximum(
        jnp.dot(emb[...], w1e[...], preferred_element_type=jnp.float32)
        + jnp.dot(xn[...], w1n[...], preferred_element_type=jnp.float32)
        + b1r[...], 0.0)
    logits = jnp.dot(h, w2[...], preferred_element_type=jnp.float32) + b2r[...]
    out[...] = logits.reshape(TB, R)


def _mlp(emb, x_num, W1, b1, W2, b2):
    return pl.pallas_call(
        _mlp_body,
        grid=(GRID,),
        in_specs=[
            pl.BlockSpec((TN, 128), lambda i: (i, 0)),
            pl.BlockSpec((TN, NUM_NUMERICAL), lambda i: (i, 0)),
            pl.BlockSpec((128, 64), lambda i: (0, 0)),
            pl.BlockSpec((NUM_NUMERICAL, 64), lambda i: (0, 0)),
            pl.BlockSpec((1, 64), lambda i: (0, 0)),
            pl.BlockSpec((64, 1), lambda i: (0, 0)),
            pl.BlockSpec((1, 1), lambda i: (0, 0)),
        ],
        out_specs=pl.BlockSpec((TB, R), lambda i: (i, 0)),
        out_shape=jax.ShapeDtypeStruct((B, R), jnp.float32),
    )(emb, jnp.reshape(x_num, (N, NUM_NUMERICAL)), W1[:128], W1[128:],
      b1.reshape(1, 64), W2, b2.reshape(1, 1))


def kernel(x_cat_going, x_cat_horse_id, x_cat_jockey_id, x_cat_race_class,
           x_cat_track_id, x_cat_trainer_id, x_num,
           table_going, table_horse_id, table_jockey_id, table_race_class,
           table_track_id, table_trainer_id, W1, b1, W2, b2):
    idx = jnp.stack([jnp.reshape(x, (NW, NCH, CHUNK)) for x in (
        x_cat_going, x_cat_horse_id, x_cat_jockey_id, x_cat_race_class,
        x_cat_track_id, x_cat_trainer_id)], axis=1)

    def band128(t, c0):
        d = t.shape[1]
        return jnp.pad(t, ((0, 0), (c0, 128 - c0 - d)))

    tabs = (table_going, table_horse_id, table_jockey_id, table_race_class,
            table_track_id, table_trainer_id)
    emb = _sc_gather(idx, *(band128(t, c0) for t, c0 in zip(tabs, COL0)))
    return _mlp(emb, x_num, W1, b1, W2, b2)
